# baseline (speedup 1.0000x reference)
import functools

import jax
import jax.numpy as jnp
from jax import lax
from jax.experimental import pallas as pl
from jax.experimental.pallas import tpu as pltpu
from jax.experimental.pallas import tpu_sc as plsc


def _emb_kernel(table_hbm, idx_hbm, out_hbm, idx_v, sem, *,
                num_cores, rows_per_worker):
    wid = lax.axis_index("s") * num_cores + lax.axis_index("c")
    base = wid * rows_per_worker
    pltpu.sync_copy(idx_hbm.at[pl.ds(base, rows_per_worker)], idx_v)

    def issue(j, carry):
        vec = idx_v[pl.ds(j * 16, 16)]
        for lane in range(16):
            row = vec[lane]
            pltpu.async_copy(
                table_hbm.at[pl.ds(row, 1)],
                out_hbm.at[pl.ds(base + j * 16 + lane, 1)],
                sem,
            )
        return carry

    lax.fori_loop(0, rows_per_worker // 16, issue, 0)

    def drain(i, carry):
        pltpu.make_async_copy(
            table_hbm.at[pl.ds(0, 1)], out_hbm.at[pl.ds(base, 1)], sem).wait()
        return carry

    lax.fori_loop(0, rows_per_worker, drain, 0)


def kernel(labels, embedding_table):
    (batch,) = labels.shape
    _, hidden = embedding_table.shape
    info = plsc.get_sparse_core_info()
    num_workers = info.num_cores * info.num_subcores
    rows_per_worker = batch // num_workers

    mesh = plsc.VectorSubcoreMesh(core_axis_name="c", subcore_axis_name="s")

    emb = pl.kernel(
        functools.partial(
            _emb_kernel,
            num_cores=info.num_cores,
            rows_per_worker=rows_per_worker,
        ),
        out_type=jax.ShapeDtypeStruct((batch, hidden), jnp.float32),
        mesh=mesh,
        scratch_types=[
            pltpu.VMEM((rows_per_worker,), jnp.int32),
            pltpu.SemaphoreType.DMA,
        ],
        compiler_params=pltpu.CompilerParams(use_tc_tiling_on_sc=True),
    )
    return emb(embedding_table, labels.astype(jnp.int32))


# speedup vs baseline: 3.6654x; 3.6654x over previous
import functools

import jax
import jax.numpy as jnp
from jax import lax
from jax.experimental import pallas as pl
from jax.experimental.pallas import tpu as pltpu
from jax.experimental.pallas import tpu_sc as plsc

_CHUNK = 128


def _emb_kernel(table_hbm, idx_hbm, out_hbm, idx_v, rows_v, sem, *,
                num_cores, rows_per_worker, hidden):
    wid = lax.axis_index("s") * num_cores + lax.axis_index("c")
    base = wid * rows_per_worker
    pltpu.sync_copy(idx_hbm.at[pl.ds(base, rows_per_worker)], idx_v)
    copies = [
        pltpu.async_copy(
            table_hbm.at[idx_v.at[pl.ds(j * _CHUNK, _CHUNK)]],
            rows_v.at[pl.ds(j * _CHUNK, _CHUNK)],
            sem,
        )
        for j in range(rows_per_worker // _CHUNK)
    ]
    for c in copies:
        c.wait()
    pltpu.sync_copy(rows_v, out_hbm.at[pl.ds(base, rows_per_worker)])


def kernel(labels, embedding_table):
    (batch,) = labels.shape
    rows, hidden = embedding_table.shape
    info = plsc.get_sparse_core_info()
    num_workers = info.num_cores * info.num_subcores
    rows_per_worker = batch // num_workers

    rpad = (-rows) % 8
    tpad = jnp.pad(embedding_table, ((0, rpad), (0, 128 - hidden)))

    mesh = plsc.VectorSubcoreMesh(core_axis_name="c", subcore_axis_name="s")

    emb = pl.kernel(
        functools.partial(
            _emb_kernel,
            num_cores=info.num_cores,
            rows_per_worker=rows_per_worker,
            hidden=hidden,
        ),
        out_type=jax.ShapeDtypeStruct((batch, 128), jnp.float32),
        mesh=mesh,
        scratch_types=[
            pltpu.VMEM((rows_per_worker,), jnp.int32),
            pltpu.VMEM((rows_per_worker, 128), jnp.float32),
            pltpu.SemaphoreType.DMA,
        ],
        compiler_params=pltpu.CompilerParams(use_tc_tiling_on_sc=True),
    )
    return emb(tpad, labels.astype(jnp.int32))[:, :hidden]
